# baseline (speedup 1.0000x reference)
import dataclasses
import functools

import jax
import jax.numpy as jnp
import numpy as np
from jax import lax
from jax.experimental import pallas as pl
from jax.experimental.pallas import tpu as pltpu
from jax.experimental.pallas import tpu_sc as plsc

N = 10000
D = 256
H = 16
DH = 16
E = 160000

NC = 2
NS = 16
CHUNK = 24
NCHUNK = 427
EDGES_PER_SUB = CHUNK * NCHUNK
E_PAD = EDGES_PER_SUB * NS
NCHT = E_PAD // CHUNK
NPAD = 10240
ZROWS = NPAD // 8
ACCR = NPAD + ZROWS
ROWS_PER_SUB = ACCR // NS

_DOT = functools.partial(jax.lax.dot_general, precision=jax.lax.Precision.HIGHEST)


def _mm(a, b):
    return _DOT(a, b, (((1,), (0,)), ((), ())), preferred_element_type=jnp.float32)


def _qkv_body(x_ref, wq_ref, wk_ref, wv_ref, q_ref, k_ref, v_ref):
    xb = x_ref[...]
    q_ref[...] = _mm(xb, wq_ref[...]).astype(jnp.bfloat16)
    k_ref[...] = _mm(xb, wk_ref[...]).astype(jnp.bfloat16)
    v_ref[...] = _mm(xb, wv_ref[...]).astype(jnp.bfloat16)


def _qkv(x_pad, wq_p, wk_p, wv_p):
    R = 2048
    grid = (NPAD // R,)
    return pl.pallas_call(
        _qkv_body,
        grid=grid,
        in_specs=[
            pl.BlockSpec((R, D), lambda i: (i, 0)),
            pl.BlockSpec((D, D), lambda i: (0, 0)),
            pl.BlockSpec((D, D), lambda i: (0, 0)),
            pl.BlockSpec((D, D), lambda i: (0, 0)),
        ],
        out_specs=[
            pl.BlockSpec((R, D), lambda i: (i, 0)),
            pl.BlockSpec((R, D), lambda i: (i, 0)),
            pl.BlockSpec((R, D), lambda i: (i, 0)),
        ],
        out_shape=[
            jax.ShapeDtypeStruct((NPAD, D), jnp.bfloat16),
            jax.ShapeDtypeStruct((NPAD, D), jnp.bfloat16),
            jax.ShapeDtypeStruct((NPAD, D), jnp.bfloat16),
        ],
    )(x_pad, wq_p, wk_p, wv_p)


def _sc_edge_body(tab_hbm, eidx_hbm, widx_hbm, zero_hbm, out_hbm,
                  ebuf, wbuf, gbuf, mbuf, sbuf, acc, gsem, ssem, isem):
    core = lax.axis_index("c")
    sid = lax.axis_index("s")
    core64 = core * (D // 4)

    pltpu.sync_copy(zero_hbm.at[pl.ds(sid * ROWS_PER_SUB, ROWS_PER_SUB)],
                    acc.at[pl.ds(sid * ROWS_PER_SUB, ROWS_PER_SUB)])
    plsc.subcore_barrier()

    row0 = sid * NCHUNK

    def idx_copies(ci, slot):
        row = jnp.minimum(row0 + ci, NCHT - 1)
        return (
            pltpu.make_async_copy(eidx_hbm.at[row], ebuf.at[slot],
                                  isem.at[slot]),
            pltpu.make_async_copy(widx_hbm.at[row], wbuf.at[slot],
                                  isem.at[slot]),
        )

    def gather_copies(slot):
        return (
            pltpu.make_async_copy(tab_hbm.at[ebuf.at[slot]], gbuf.at[slot],
                                  gsem.at[slot]),
        )

    def scatter_copies(slot):
        return (
            pltpu.make_async_copy(mbuf.at[slot], acc.at[wbuf.at[slot]],
                                  ssem.at[slot]),
        )

    def start_scatters(slot):
        pltpu.async_copy(mbuf.at[slot], acc.at[wbuf.at[slot]],
                         ssem.at[slot], add=True)

    def issue_gathers(slot):
        for c in gather_copies(slot):
            c.start()

    def body(ci, p, first=False, last=False):
        for c in gather_copies(p):
            c.wait()
        for c in idx_copies(ci + 1, 1 - p):
            c.wait()
        if not last:
            issue_gathers(1 - p)
        if not first:
            @pl.when(ci >= 2)
            def _():
                for c in scatter_copies(p):
                    c.wait()

        def bf2(base, e, gg):
            w = gbuf[p, base + e, pl.ds(gg * 16, 16)]
            a = plsc.bitcast(w << 16, jnp.float32)
            b = plsc.bitcast(w, jnp.float32)
            return a, b

        @plsc.parallel_loop(0, CHUNK // 2, 1, unroll=2)
        def _pair(pi):
            for r in range(2):
                e = pi * 2 + r
                t = [None] * 4
                for gg in range(8):
                    ka, kb = bf2(0, e, gg)
                    qa, qb = bf2(CHUNK, e, gg)
                    c = gg & 3
                    pa = ka * qa
                    pb = kb * qb
                    if t[c] is None:
                        t[c] = pa + pb
                    else:
                        t[c] = t[c] + (pa + pb)
                s = (t[0] + t[1]) + (t[2] + t[3])
                s = jnp.minimum(jnp.maximum(s, -5.0), 5.0)
                s = jnp.exp(s)
                sbuf[p, e] = s
                for gv in range(4):
                    w = gbuf[p, 2 * CHUNK + e, pl.ds(core64 + gv * 16, 16)]
                    va = plsc.bitcast(w << 16, jnp.float32)
                    vb = plsc.bitcast(w, jnp.float32)
                    mbuf[p, e, pl.ds((2 * gv) * 16, 16)] = va * s
                    mbuf[p, e, pl.ds((2 * gv + 1) * 16, 16)] = vb * s

        zero16 = jnp.zeros((16,), jnp.float32)
        for j in range(2):
            nlanes = 16 if j == 0 else CHUNK - 16
            offv = (wbuf[p, pl.ds(j * 16, 16)] & 7) * 16
            for e2 in range(nlanes):
                e = j * 16 + e2
                for gz in range(8):
                    mbuf[p, CHUNK + e, pl.ds(gz * 16, 16)] = zero16
                mbuf[p, CHUNK + e, pl.ds(offv[e2], 16)] = sbuf[p, e]

        start_scatters(p)
        if not last:
            for c in idx_copies(ci + 2, p):
                c.start()

    for c in idx_copies(0, 0):
        c.start()
    for c in idx_copies(0, 0):
        c.wait()
    issue_gathers(0)
    for c in idx_copies(1, 1):
        c.start()

    @pl.loop(0, (NCHUNK - 1) // 2)
    def _pair_loop(g):
        body(2 * g, 0, first=False)
        body(2 * g + 1, 1)

    body(NCHUNK - 1, 0, last=True)

    for c in scatter_copies(1):
        c.wait()
    for c in scatter_copies(0):
        c.wait()

    plsc.subcore_barrier()
    pltpu.sync_copy(acc.at[pl.ds(sid * ROWS_PER_SUB, ROWS_PER_SUB)],
                    out_hbm.at[core, pl.ds(sid * ROWS_PER_SUB, ROWS_PER_SUB)])


def _sc_edge(tab, eidx, widx, zeros_init):
    mesh = plsc.VectorSubcoreMesh(core_axis_name="c", subcore_axis_name="s")
    cp = pltpu.CompilerParams()
    if "needs_layout_passes" in pltpu.CompilerParams.__dataclass_fields__:
        cp = dataclasses.replace(cp, needs_layout_passes=False)
    run = pl.kernel(
        _sc_edge_body,
        compiler_params=cp,
        out_type=jax.ShapeDtypeStruct((2, ACCR, 128), jnp.float32),
        mesh=mesh,
        scratch_types=[
            pltpu.VMEM((2, 3 * CHUNK), jnp.int32),
            pltpu.VMEM((2, 2 * CHUNK), jnp.int32),
            pltpu.VMEM((2, 3 * CHUNK, 128), jnp.int32),
            pltpu.VMEM((2, 2 * CHUNK, 128), jnp.float32),
            pltpu.VMEM((2, CHUNK, 16), jnp.float32),
            pltpu.VMEM_SHARED((ACCR, 128), jnp.float32),
            pltpu.SemaphoreType.DMA((2,)),
            pltpu.SemaphoreType.DMA((2,)),
            pltpu.SemaphoreType.DMA((2,)),
        ],
    )
    return run(tab, eidx, widx, zeros_init)


def _attn_out_body(x_ref, wv_ref, z_ref, wo_ref, h1_ref, st_ref):
    i = pl.program_id(0)
    wv = jnp.concatenate([wv_ref[0], wv_ref[1]], axis=1)
    denom = jnp.concatenate([z_ref[...]] * (D // DH), axis=1) + 1e-6
    h1 = x_ref[...] + _mm(wv / denom, wo_ref[...])
    h1_ref[...] = h1

    s = jnp.sum(h1, axis=0, keepdims=True)
    q = jnp.sum(h1 * h1, axis=0, keepdims=True)
    upd = jnp.concatenate([s, q, jnp.zeros((6, D), jnp.float32)], axis=0)

    @pl.when(i == 0)
    def _():
        st_ref[...] = jnp.zeros_like(st_ref)

    st_ref[...] += upd


def _attn_out(x, wvacc, z, wo_t):
    R = 2000
    return pl.pallas_call(
        _attn_out_body,
        grid=(N // R,),
        in_specs=[
            pl.BlockSpec((R, D), lambda i: (i, 0)),
            pl.BlockSpec((2, R, D // 2), lambda i: (0, i, 0)),
            pl.BlockSpec((R, DH), lambda i: (i, 0)),
            pl.BlockSpec((D, D), lambda i: (0, 0)),
        ],
        out_specs=[
            pl.BlockSpec((R, D), lambda i: (i, 0)),
            pl.BlockSpec((8, D), lambda i: (0, 0)),
        ],
        out_shape=[
            jax.ShapeDtypeStruct((N, D), jnp.float32),
            jax.ShapeDtypeStruct((8, D), jnp.float32),
        ],
    )(x, wvacc, z, wo_t)


def _ffn_body(h1_ref, st1_ref, g1_ref, b1n_ref, w1_ref, bb1_ref, w2_ref,
              bb2_ref, h2_ref, st2_ref):
    i = pl.program_id(0)
    inv_n = jnp.float32(1.0 / N)
    mu = st1_ref[0:1, :] * inv_n
    var = st1_ref[1:2, :] * inv_n - mu * mu
    rstd = jax.lax.rsqrt(var + 1e-5)
    h1n = g1_ref[...] * (h1_ref[...] - mu) * rstd + b1n_ref[...]
    hid = jnp.maximum(_mm(h1n, w1_ref[...]) + bb1_ref[...], 0.0)
    h2 = h1n + _mm(hid, w2_ref[...]) + bb2_ref[...]
    h2_ref[...] = h2

    s = jnp.sum(h2, axis=0, keepdims=True)
    q = jnp.sum(h2 * h2, axis=0, keepdims=True)
    upd = jnp.concatenate([s, q, jnp.zeros((6, D), jnp.float32)], axis=0)

    @pl.when(i == 0)
    def _():
        st2_ref[...] = jnp.zeros_like(st2_ref)

    st2_ref[...] += upd


def _ffn(h1, st1, gamma1, beta1, w1, b1, w2, b2):
    R = 2000
    return pl.pallas_call(
        _ffn_body,
        grid=(N // R,),
        in_specs=[
            pl.BlockSpec((R, D), lambda i: (i, 0)),
            pl.BlockSpec((8, D), lambda i: (0, 0)),
            pl.BlockSpec((1, D), lambda i: (0, 0)),
            pl.BlockSpec((1, D), lambda i: (0, 0)),
            pl.BlockSpec((D, 2 * D), lambda i: (0, 0)),
            pl.BlockSpec((1, 2 * D), lambda i: (0, 0)),
            pl.BlockSpec((2 * D, D), lambda i: (0, 0)),
            pl.BlockSpec((1, D), lambda i: (0, 0)),
        ],
        out_specs=[
            pl.BlockSpec((R, D), lambda i: (i, 0)),
            pl.BlockSpec((8, D), lambda i: (0, 0)),
        ],
        out_shape=[
            jax.ShapeDtypeStruct((N, D), jnp.float32),
            jax.ShapeDtypeStruct((8, D), jnp.float32),
        ],
    )(h1, st1, gamma1, beta1, w1, b1, w2, b2)


def _bn2_body(h2_ref, st2_ref, g2_ref, b2n_ref, out_ref):
    inv_n = jnp.float32(1.0 / N)
    mu = st2_ref[0:1, :] * inv_n
    var = st2_ref[1:2, :] * inv_n - mu * mu
    rstd = jax.lax.rsqrt(var + 1e-5)
    out_ref[...] = g2_ref[...] * (h2_ref[...] - mu) * rstd + b2n_ref[...]


def _bn2(h2, st2, gamma2, beta2):
    R = 2000
    return pl.pallas_call(
        _bn2_body,
        grid=(N // R,),
        in_specs=[
            pl.BlockSpec((R, D), lambda i: (i, 0)),
            pl.BlockSpec((8, D), lambda i: (0, 0)),
            pl.BlockSpec((1, D), lambda i: (0, 0)),
            pl.BlockSpec((1, D), lambda i: (0, 0)),
        ],
        out_specs=pl.BlockSpec((R, D), lambda i: (i, 0)),
        out_shape=jax.ShapeDtypeStruct((N, D), jnp.float32),
    )(h2, st2, gamma2, beta2)


@jax.jit
def kernel(x, edge_index, Wq, Wk, Wv, Wo, gamma1, beta1, W1, b1, W2, b2,
           gamma2, beta2):
    perm_kq = np.empty((D,), np.int32)
    for d in range(DH):
        for h in range(H):
            perm_kq[32 * (d // 2) + 2 * h + (d % 2)] = h * 16 + d
    perm_v = np.empty((D,), np.int32)
    for hv in range(2):
        for dp in range(8):
            for h in range(H):
                perm_v[128 * hv + 32 * (dp // 2) + 2 * h + (dp % 2)] = (
                    h * 16 + hv * 8 + dp)

    wq_p = jnp.take(Wq, jnp.asarray(perm_kq), axis=1) * 0.25
    wk_p = jnp.take(Wk, jnp.asarray(perm_kq), axis=1)
    wv_p = jnp.take(Wv, jnp.asarray(perm_v), axis=1)
    wo_t = Wo.reshape(H, DH, D).transpose(1, 0, 2).reshape(D, D)

    src = edge_index[0]
    dst = edge_index[1]
    npad_e = E_PAD - E
    src_p = jnp.concatenate([src, jnp.zeros((npad_e,), jnp.int32)])
    dst_p = jnp.concatenate([dst, jnp.full((npad_e,), N, jnp.int32)])
    x_pad = jnp.concatenate([x, jnp.zeros((NPAD - N, D), x.dtype)], axis=0)

    qtb, ktb, vtb = _qkv(x_pad, wq_p, wk_p, wv_p)
    def pack32(a):
        return jax.lax.bitcast_convert_type(
            a.reshape(a.shape[:-1] + (a.shape[-1] // 2, 2)), jnp.int32)

    tab = jnp.concatenate([pack32(ktb), pack32(qtb), pack32(vtb)], axis=0)

    srcr = src_p.reshape(NCHT, CHUNK)
    dstr = dst_p.reshape(NCHT, CHUNK)
    eidx = jnp.concatenate([srcr, dstr + NPAD, srcr + 2 * NPAD], axis=1)
    widx = jnp.concatenate(
        [dstr, jax.lax.shift_right_logical(dstr, 3) + NPAD], axis=1)

    zeros_init = jnp.zeros((ACCR, 128), jnp.float32)
    out = _sc_edge(tab, eidx, widx, zeros_init)
    wvacc = out[:, :N, :]
    z = out[0, NPAD:].reshape(NPAD, DH)[:N]

    h1, st1 = _attn_out(x, wvacc, z, wo_t)
    h2, st2 = _ffn(h1, st1, gamma1.reshape(1, D), beta1.reshape(1, D),
                   W1, b1.reshape(1, 2 * D), W2, b2.reshape(1, D))
    return _bn2(h2, st2, gamma2.reshape(1, D), beta2.reshape(1, D))


# speedup vs baseline: 1.0300x; 1.0300x over previous
import dataclasses
import functools

import jax
import jax.numpy as jnp
import numpy as np
from jax import lax
from jax.experimental import pallas as pl
from jax.experimental.pallas import tpu as pltpu
from jax.experimental.pallas import tpu_sc as plsc

N = 10000
D = 256
H = 16
DH = 16
E = 160000

NC = 2
NS = 16
CHUNK = 24
NCHUNK = 427
EDGES_PER_SUB = CHUNK * NCHUNK
E_PAD = EDGES_PER_SUB * NS
NCHT = E_PAD // CHUNK
NPAD = 10240
ZROWS = NPAD // 8
ACCR = NPAD + ZROWS
ROWS_PER_SUB = ACCR // NS

_DOT = functools.partial(jax.lax.dot_general, precision=jax.lax.Precision.HIGHEST)


def _mm(a, b):
    return _DOT(a, b, (((1,), (0,)), ((), ())), preferred_element_type=jnp.float32)


def _qkv_body(x_ref, wq_ref, wk_ref, wv_ref, q_ref, k_ref, v_ref):
    xb = x_ref[...]
    q_ref[...] = _mm(xb, wq_ref[...]).astype(jnp.bfloat16)
    k_ref[...] = _mm(xb, wk_ref[...]).astype(jnp.bfloat16)
    v_ref[...] = _mm(xb, wv_ref[...]).astype(jnp.bfloat16)


def _qkv(x_pad, wq_p, wk_p, wv_p):
    R = 2048
    grid = (NPAD // R,)
    return pl.pallas_call(
        _qkv_body,
        grid=grid,
        in_specs=[
            pl.BlockSpec((R, D), lambda i: (i, 0)),
            pl.BlockSpec((D, D), lambda i: (0, 0)),
            pl.BlockSpec((D, D), lambda i: (0, 0)),
            pl.BlockSpec((D, D), lambda i: (0, 0)),
        ],
        out_specs=[
            pl.BlockSpec((R, D), lambda i: (i, 0)),
            pl.BlockSpec((R, D), lambda i: (i, 0)),
            pl.BlockSpec((R, D), lambda i: (i, 0)),
        ],
        out_shape=[
            jax.ShapeDtypeStruct((NPAD, D), jnp.bfloat16),
            jax.ShapeDtypeStruct((NPAD, D), jnp.bfloat16),
            jax.ShapeDtypeStruct((NPAD, D), jnp.bfloat16),
        ],
    )(x_pad, wq_p, wk_p, wv_p)


def _sc_edge_body(tab_hbm, eidx_hbm, widx_hbm, zero_hbm, out_hbm,
                  ebuf, wbuf, gbuf, mbuf, sbuf, acc, gsem, ssem, isem):
    core = lax.axis_index("c")
    sid = lax.axis_index("s")
    core64 = core * (D // 4)

    pltpu.sync_copy(zero_hbm.at[pl.ds(sid * ROWS_PER_SUB, ROWS_PER_SUB)],
                    acc.at[pl.ds(sid * ROWS_PER_SUB, ROWS_PER_SUB)])
    plsc.subcore_barrier()

    row0 = sid * NCHUNK

    def idx_copies(ci, slot):
        row = jnp.minimum(row0 + ci, NCHT - 1)
        return (
            pltpu.make_async_copy(eidx_hbm.at[row], ebuf.at[slot],
                                  isem.at[slot]),
            pltpu.make_async_copy(widx_hbm.at[row], wbuf.at[slot],
                                  isem.at[slot]),
        )

    def gather_copies(slot):
        return (
            pltpu.make_async_copy(tab_hbm.at[ebuf.at[slot]], gbuf.at[slot],
                                  gsem.at[slot]),
        )

    def scatter_copies(slot):
        return (
            pltpu.make_async_copy(mbuf.at[slot], acc.at[wbuf.at[slot]],
                                  ssem.at[slot]),
        )

    def start_scatters(slot):
        pltpu.async_copy(mbuf.at[slot], acc.at[wbuf.at[slot]],
                         ssem.at[slot], add=True)

    def issue_gathers(slot):
        for c in gather_copies(slot):
            c.start()

    def body(ci, p, first=False, last=False):
        for c in gather_copies(p):
            c.wait()
        for c in idx_copies(ci + 1, 1 - p):
            c.wait()
        if not last:
            issue_gathers(1 - p)
        if not first:
            @pl.when(ci >= 2)
            def _():
                for c in scatter_copies(p):
                    c.wait()

        def bf2(base, e, gg):
            w = gbuf[p, base + e, pl.ds(gg * 16, 16)]
            a = plsc.bitcast(w << 16, jnp.float32)
            b = plsc.bitcast(w, jnp.float32)
            return a, b

        @pl.loop(0, CHUNK // 2)
        def _pair(pi):
            for r in range(2):
                e = pi * 2 + r
                t = [None] * 4
                for gg in range(8):
                    ka, kb = bf2(0, e, gg)
                    qa, qb = bf2(CHUNK, e, gg)
                    c = gg & 3
                    pa = ka * qa
                    pb = kb * qb
                    if t[c] is None:
                        t[c] = pa + pb
                    else:
                        t[c] = t[c] + (pa + pb)
                s = (t[0] + t[1]) + (t[2] + t[3])
                s = s * 0.25
                s = jnp.minimum(jnp.maximum(s, -5.0), 5.0)
                s = jnp.exp(s)
                sbuf[p, e] = s
                for gv in range(4):
                    w = gbuf[p, 2 * CHUNK + e, pl.ds(core64 + gv * 16, 16)]
                    va = plsc.bitcast(w << 16, jnp.float32)
                    vb = plsc.bitcast(w, jnp.float32)
                    mbuf[p, e, pl.ds((2 * gv) * 16, 16)] = va * s
                    mbuf[p, e, pl.ds((2 * gv + 1) * 16, 16)] = vb * s

        zero16 = jnp.zeros((16,), jnp.float32)
        for j in range(2):
            nlanes = 16 if j == 0 else CHUNK - 16
            offv = (wbuf[p, pl.ds(j * 16, 16)] & 7) * 16
            for e2 in range(nlanes):
                e = j * 16 + e2
                for gz in range(8):
                    mbuf[p, CHUNK + e, pl.ds(gz * 16, 16)] = zero16
                mbuf[p, CHUNK + e, pl.ds(offv[e2], 16)] = sbuf[p, e]

        start_scatters(p)
        if not last:
            for c in idx_copies(ci + 2, p):
                c.start()

    for c in idx_copies(0, 0):
        c.start()
    for c in idx_copies(0, 0):
        c.wait()
    issue_gathers(0)
    for c in idx_copies(1, 1):
        c.start()

    @pl.loop(0, (NCHUNK - 1) // 2)
    def _pair_loop(g):
        body(2 * g, 0, first=False)
        body(2 * g + 1, 1)

    body(NCHUNK - 1, 0, last=True)

    for c in scatter_copies(1):
        c.wait()
    for c in scatter_copies(0):
        c.wait()

    plsc.subcore_barrier()
    pltpu.sync_copy(acc.at[pl.ds(sid * ROWS_PER_SUB, ROWS_PER_SUB)],
                    out_hbm.at[core, pl.ds(sid * ROWS_PER_SUB, ROWS_PER_SUB)])


def _sc_edge(tab, eidx, widx, zeros_init):
    mesh = plsc.VectorSubcoreMesh(core_axis_name="c", subcore_axis_name="s")
    cp = pltpu.CompilerParams()
    if "needs_layout_passes" in pltpu.CompilerParams.__dataclass_fields__:
        cp = dataclasses.replace(cp, needs_layout_passes=False)
    run = pl.kernel(
        _sc_edge_body,
        compiler_params=cp,
        out_type=jax.ShapeDtypeStruct((2, ACCR, 128), jnp.float32),
        mesh=mesh,
        scratch_types=[
            pltpu.VMEM((2, 3 * CHUNK), jnp.int32),
            pltpu.VMEM((2, 2 * CHUNK), jnp.int32),
            pltpu.VMEM((2, 3 * CHUNK, 128), jnp.int32),
            pltpu.VMEM((2, 2 * CHUNK, 128), jnp.float32),
            pltpu.VMEM((2, CHUNK, 16), jnp.float32),
            pltpu.VMEM_SHARED((ACCR, 128), jnp.float32),
            pltpu.SemaphoreType.DMA((2,)),
            pltpu.SemaphoreType.DMA((2,)),
            pltpu.SemaphoreType.DMA((2,)),
        ],
    )
    return run(tab, eidx, widx, zeros_init)


def _attn_out_body(x_ref, wv_ref, z_ref, wo_ref, h1_ref, st_ref):
    i = pl.program_id(0)
    wv = jnp.concatenate([wv_ref[0], wv_ref[1]], axis=1)
    denom = jnp.concatenate([z_ref[...]] * (D // DH), axis=1) + 1e-6
    h1 = x_ref[...] + _mm(wv / denom, wo_ref[...])
    h1_ref[...] = h1

    s = jnp.sum(h1, axis=0, keepdims=True)
    q = jnp.sum(h1 * h1, axis=0, keepdims=True)
    upd = jnp.concatenate([s, q, jnp.zeros((6, D), jnp.float32)], axis=0)

    @pl.when(i == 0)
    def _():
        st_ref[...] = jnp.zeros_like(st_ref)

    st_ref[...] += upd


def _attn_out(x, wvacc, z, wo_t):
    R = 2000
    return pl.pallas_call(
        _attn_out_body,
        grid=(N // R,),
        in_specs=[
            pl.BlockSpec((R, D), lambda i: (i, 0)),
            pl.BlockSpec((2, R, D // 2), lambda i: (0, i, 0)),
            pl.BlockSpec((R, DH), lambda i: (i, 0)),
            pl.BlockSpec((D, D), lambda i: (0, 0)),
        ],
        out_specs=[
            pl.BlockSpec((R, D), lambda i: (i, 0)),
            pl.BlockSpec((8, D), lambda i: (0, 0)),
        ],
        out_shape=[
            jax.ShapeDtypeStruct((N, D), jnp.float32),
            jax.ShapeDtypeStruct((8, D), jnp.float32),
        ],
    )(x, wvacc, z, wo_t)


def _ffn_body(h1_ref, st1_ref, g1_ref, b1n_ref, w1_ref, bb1_ref, w2_ref,
              bb2_ref, h2_ref, st2_ref):
    i = pl.program_id(0)
    inv_n = jnp.float32(1.0 / N)
    mu = st1_ref[0:1, :] * inv_n
    var = st1_ref[1:2, :] * inv_n - mu * mu
    rstd = jax.lax.rsqrt(var + 1e-5)
    h1n = g1_ref[...] * (h1_ref[...] - mu) * rstd + b1n_ref[...]
    hid = jnp.maximum(_mm(h1n, w1_ref[...]) + bb1_ref[...], 0.0)
    h2 = h1n + _mm(hid, w2_ref[...]) + bb2_ref[...]
    h2_ref[...] = h2

    s = jnp.sum(h2, axis=0, keepdims=True)
    q = jnp.sum(h2 * h2, axis=0, keepdims=True)
    upd = jnp.concatenate([s, q, jnp.zeros((6, D), jnp.float32)], axis=0)

    @pl.when(i == 0)
    def _():
        st2_ref[...] = jnp.zeros_like(st2_ref)

    st2_ref[...] += upd


def _ffn(h1, st1, gamma1, beta1, w1, b1, w2, b2):
    R = 2000
    return pl.pallas_call(
        _ffn_body,
        grid=(N // R,),
        in_specs=[
            pl.BlockSpec((R, D), lambda i: (i, 0)),
            pl.BlockSpec((8, D), lambda i: (0, 0)),
            pl.BlockSpec((1, D), lambda i: (0, 0)),
            pl.BlockSpec((1, D), lambda i: (0, 0)),
            pl.BlockSpec((D, 2 * D), lambda i: (0, 0)),
            pl.BlockSpec((1, 2 * D), lambda i: (0, 0)),
            pl.BlockSpec((2 * D, D), lambda i: (0, 0)),
            pl.BlockSpec((1, D), lambda i: (0, 0)),
        ],
        out_specs=[
            pl.BlockSpec((R, D), lambda i: (i, 0)),
            pl.BlockSpec((8, D), lambda i: (0, 0)),
        ],
        out_shape=[
            jax.ShapeDtypeStruct((N, D), jnp.float32),
            jax.ShapeDtypeStruct((8, D), jnp.float32),
        ],
    )(h1, st1, gamma1, beta1, w1, b1, w2, b2)


def _bn2_body(h2_ref, st2_ref, g2_ref, b2n_ref, out_ref):
    inv_n = jnp.float32(1.0 / N)
    mu = st2_ref[0:1, :] * inv_n
    var = st2_ref[1:2, :] * inv_n - mu * mu
    rstd = jax.lax.rsqrt(var + 1e-5)
    out_ref[...] = g2_ref[...] * (h2_ref[...] - mu) * rstd + b2n_ref[...]


def _bn2(h2, st2, gamma2, beta2):
    R = 2000
    return pl.pallas_call(
        _bn2_body,
        grid=(N // R,),
        in_specs=[
            pl.BlockSpec((R, D), lambda i: (i, 0)),
            pl.BlockSpec((8, D), lambda i: (0, 0)),
            pl.BlockSpec((1, D), lambda i: (0, 0)),
            pl.BlockSpec((1, D), lambda i: (0, 0)),
        ],
        out_specs=pl.BlockSpec((R, D), lambda i: (i, 0)),
        out_shape=jax.ShapeDtypeStruct((N, D), jnp.float32),
    )(h2, st2, gamma2, beta2)


@jax.jit
def kernel(x, edge_index, Wq, Wk, Wv, Wo, gamma1, beta1, W1, b1, W2, b2,
           gamma2, beta2):
    perm_kq = np.empty((D,), np.int32)
    for d in range(DH):
        for h in range(H):
            perm_kq[32 * (d // 2) + 2 * h + (d % 2)] = h * 16 + d
    perm_v = np.empty((D,), np.int32)
    for hv in range(2):
        for dp in range(8):
            for h in range(H):
                perm_v[128 * hv + 32 * (dp // 2) + 2 * h + (dp % 2)] = (
                    h * 16 + hv * 8 + dp)

    wq_p = jnp.take(Wq, jnp.asarray(perm_kq), axis=1)
    wk_p = jnp.take(Wk, jnp.asarray(perm_kq), axis=1)
    wv_p = jnp.take(Wv, jnp.asarray(perm_v), axis=1)
    wo_t = Wo.reshape(H, DH, D).transpose(1, 0, 2).reshape(D, D)

    src = edge_index[0]
    dst = edge_index[1]
    npad_e = E_PAD - E
    src_p = jnp.concatenate([src, jnp.zeros((npad_e,), jnp.int32)])
    dst_p = jnp.concatenate([dst, jnp.full((npad_e,), N, jnp.int32)])
    x_pad = jnp.concatenate([x, jnp.zeros((NPAD - N, D), x.dtype)], axis=0)

    qtb, ktb, vtb = _qkv(x_pad, wq_p, wk_p, wv_p)
    def pack32(a):
        return jax.lax.bitcast_convert_type(
            a.reshape(a.shape[:-1] + (a.shape[-1] // 2, 2)), jnp.int32)

    tab = jnp.concatenate([pack32(ktb), pack32(qtb), pack32(vtb)], axis=0)

    srcr = src_p.reshape(NCHT, CHUNK)
    dstr = dst_p.reshape(NCHT, CHUNK)
    eidx = jnp.concatenate([srcr, dstr + NPAD, srcr + 2 * NPAD], axis=1)
    widx = jnp.concatenate(
        [dstr, jax.lax.shift_right_logical(dstr, 3) + NPAD], axis=1)

    zeros_init = jnp.zeros((ACCR, 128), jnp.float32)
    out = _sc_edge(tab, eidx, widx, zeros_init)
    wvacc = out[:, :N, :]
    z = out[0, NPAD:].reshape(NPAD, DH)[:N]

    h1, st1 = _attn_out(x, wvacc, z, wo_t)
    h2, st2 = _ffn(h1, st1, gamma1.reshape(1, D), beta1.reshape(1, D),
                   W1, b1.reshape(1, 2 * D), W2, b2.reshape(1, D))
    return _bn2(h2, st2, gamma2.reshape(1, D), beta2.reshape(1, D))
